# baseline (speedup 1.0000x reference)
import functools

import jax
import jax.numpy as jnp
from jax.experimental import pallas as pl
from jax.experimental.pallas import tpu as pltpu

_N = 4096
_D = 64
_K = 4
_BN = 512


def _mixed_op_body(mask_ref, h_ref, w_ref, out_ref, hc_ref):
    @pl.when(pl.program_id(0) == 0)
    def _combine():
        acc = jnp.zeros((_N, _D), jnp.float32)
        for k in range(_K):
            wk = w_ref[k]
            acc = acc + jnp.where(wk > 0, wk, 0.0) * h_ref[k]
        hc_ref[...] = acc

    c = jnp.float32(0.0)
    for k in range(_K):
        wk = w_ref[k]
        c = c + jnp.where(wk > 0, jnp.float32(0.0), wk)
    out_ref[...] = (
        jnp.dot(mask_ref[...], hc_ref[...], preferred_element_type=jnp.float32) + c
    )


@jax.jit
def kernel(mask_matrix, h_op_list, weights):
    return pl.pallas_call(
        _mixed_op_body,
        grid=(_N // _BN,),
        in_specs=[
            pl.BlockSpec((_BN, _N), lambda i: (i, 0)),
            pl.BlockSpec((_K, _N, _D), lambda i: (0, 0, 0)),
            pl.BlockSpec(memory_space=pltpu.SMEM),
        ],
        out_specs=pl.BlockSpec((_BN, _D), lambda i: (i, 0)),
        out_shape=jax.ShapeDtypeStruct((_N, _D), jnp.float32),
        scratch_shapes=[pltpu.VMEM((_N, _D), jnp.float32)],
    )(mask_matrix, h_op_list, weights)
